# baseline (speedup 1.0000x reference)
import jax
import jax.numpy as jnp
from jax import lax
from jax.experimental import pallas as pl
from jax.experimental.pallas import tpu as pltpu, tpu_sc as plsc

_NC, _NS, _LANES = 2, 16, 16
_NW = _NC * _NS

_B, _L, _D = 4, 2048, 1024
_LPW = _L // _NW
_CROWS = 16
_CW = _CROWS * _D
_NCH = _B * (_LPW // _CROWS)
_NXB = 4
_NPB = 2


def _sc_body(x_hbm, pe_hbm, o_hbm, xbuf_all, pbuf_all, sxs, sps, sos):
    xbufs = [xbuf_all.at[pl.ds(_CROWS * i, _CROWS)] for i in range(_NXB)]
    pbufs = [pbuf_all.at[pl.ds((_CROWS + 8) * i, _CROWS + 8)]
             for i in range(_NPB)]
    wid = lax.axis_index("s") * _NC + lax.axis_index("c")
    lbase = wid * _LPW
    nh = _LPW // _CROWS

    def x_slice(k):
        b, h = k % _B, k // _B
        row = pl.multiple_of(b * _L + lbase + h * _CROWS, 8)
        return pl.ds(row, _CROWS)

    def start_x(k):
        return pltpu.async_copy(x_hbm.at[x_slice(k)], xbufs[k % _NXB],
                                sxs.at[k % _NXB])

    def pe_slice(h):
        row = pl.multiple_of(lbase + h * _CROWS, 8)
        return pl.ds(row, _CROWS + 8)

    def start_pe(h, slot):
        pltpu.async_copy(pe_hbm.at[pe_slice(h)], pbufs[slot], sps.at[slot])

    start_pe(0, 0)
    start_pe(1, 1)
    start_x(0)
    start_x(1)

    def half(g, carry):
        for j in range(8):
            k = g * 8 + j
            if j % 4 == 0:
                slot = j // 4
                pltpu.make_async_copy(pe_hbm.at[pe_slice(2 * g + slot)],
                                      pbufs[slot], sps.at[slot]).wait()
            pltpu.make_async_copy(x_hbm.at[x_slice(k)], xbufs[j % 4],
                                  sxs.at[j % 4]).wait()
            xbuf, pbuf = xbufs[j % 4], pbufs[j // 4]
            for r in range(_CROWS):

                @plsc.parallel_loop(0, _D, step=_LANES, unroll=8)
                def _add(c):
                    xbuf[r, pl.ds(c, _LANES)] = (
                        xbuf[r, pl.ds(c, _LANES)]
                        + pbuf[r + 1, pl.ds(c, _LANES)])

            pltpu.async_copy(xbuf, o_hbm.at[x_slice(k)], sos.at[j % 4])
            nslot = (j + 2) % 4

            def wait_prev_out():
                pltpu.make_async_copy(xbufs[nslot], o_hbm.at[x_slice(k)],
                                      sos.at[nslot]).wait()

            if j < 2:
                pl.when(g > 0)(wait_prev_out)
                pltpu.async_copy(x_hbm.at[x_slice(k + 2)], xbufs[nslot],
                                 sxs.at[nslot])
            elif j < 6:
                wait_prev_out()
                pltpu.async_copy(x_hbm.at[x_slice(k + 2)], xbufs[nslot],
                                 sxs.at[nslot])
            else:
                @pl.when(g < 1)
                def _():
                    wait_prev_out()
                    pltpu.async_copy(x_hbm.at[x_slice(k + 2)], xbufs[nslot],
                                     sxs.at[nslot])
            if j % 4 == 3:
                slot = j // 4

                @pl.when(g < 1)
                def _():
                    start_pe(2 * g + 2 + slot, slot)
        return carry

    lax.fori_loop(0, 2, half, 0)
    for j in range(4):
        pltpu.make_async_copy(xbufs[j], o_hbm.at[x_slice(12 + j)],
                              sos.at[j]).wait()


def _sc_kernel(x, pe):
    b, l, d = x.shape
    mesh = plsc.VectorSubcoreMesh(
        core_axis_name="c", subcore_axis_name="s",
        num_cores=_NC, num_subcores=_NS)
    out = pl.kernel(
        _sc_body,
        out_type=jax.ShapeDtypeStruct((b * l, d), x.dtype),
        mesh=mesh,
        scratch_types=[
            pltpu.VMEM((_CROWS * _NXB, _D), jnp.float32),
            pltpu.VMEM(((_CROWS + 8) * _NPB, _D), jnp.float32),
            pltpu.SemaphoreType.DMA((_NXB,)),
            pltpu.SemaphoreType.DMA((_NPB,)),
            pltpu.SemaphoreType.DMA((_NXB,)),
        ],
    )(x.reshape(b * l, d), pe)
    return out.reshape(b, l, d)


def kernel(x, pe):
    return _sc_kernel(x, pe)
